# baseline (speedup 1.0000x reference)
import functools

import jax
import jax.numpy as jnp
from jax.experimental import pallas as pl
from jax.experimental.pallas import tpu as pltpu
from jax.experimental.pallas import tpu_sc as plsc

_DT = 0.001
_N = 100000
_D = 128
_L = 16


def _interp_body(t_hbm, table_hbm, out_hbm, t_v, rows_v, out_v, sem):
    pltpu.sync_copy(t_hbm, t_v.at[pl.ds(0, 1)])
    x = (t_v[pl.ds(0, _L)] / jnp.float32(_DT))[0]
    trunc = x.astype(jnp.int32)
    last_id = jnp.where(x < trunc.astype(jnp.float32), trunc - 1, trunc)
    w = (last_id + 1).astype(jnp.float32) - x
    last_c = jnp.clip(last_id, 0, _N - 1)
    next_c = jnp.clip(last_id + 1, 0, _N - 1)
    cp_a = pltpu.async_copy(
        table_hbm.at[pl.ds(last_c, 1)], rows_v.at[pl.ds(0, 1)], sem
    )
    cp_b = pltpu.async_copy(
        table_hbm.at[pl.ds(next_c, 1)], rows_v.at[pl.ds(1, 1)], sem
    )
    cp_a.wait()
    cp_b.wait()
    for i in range(_D // _L):
        a = rows_v[0, pl.ds(i * _L, _L)]
        b = rows_v[1, pl.ds(i * _L, _L)]
        out_v[pl.ds(i * _L, _L)] = w * a + (jnp.float32(1.0) - w) * b
    pltpu.sync_copy(out_v, out_hbm)


_interp = functools.partial(
    pl.kernel,
    out_type=jax.ShapeDtypeStruct((_D,), jnp.float32),
    mesh=plsc.VectorSubcoreMesh(
        core_axis_name="c", subcore_axis_name="s", num_cores=1, num_subcores=1
    ),
    scratch_types=[
        pltpu.VMEM((_L,), jnp.float32),
        pltpu.VMEM((2, _D), jnp.float32),
        pltpu.VMEM((_D,), jnp.float32),
        pltpu.SemaphoreType.DMA,
    ],
)(_interp_body)


def kernel(t, excitation_data):
    return _interp(t.reshape(1), excitation_data)
